# Initial kernel scaffold; baseline (speedup 1.0000x reference)
import jax
import jax.numpy as jnp
from jax.experimental import pallas as pl


def kernel(cg_z, xyz, cg_xyz, bond_edge_list, CG_nbr_list, cg_map, ca_idx, emb, W1, Wg):
    raise NotImplementedError("write your pallas kernel here")


# speedup vs baseline: 30.0748x; 30.0748x over previous
import functools

import jax
import jax.numpy as jnp
from jax import lax
from jax.experimental import pallas as pl
from jax.experimental.pallas import tpu as pltpu
from jax.experimental.pallas import tpu_sc as plsc

N_CG = 10000
N_CG_PAD = 10240
N_ATOM = 80000
N_ATOM_PAD = 81920
E_CG = 320000
D = 128
C = 32

NC = 2
NS = 16
NW = NC * NS
L = 16

BN = 1280
E_CHUNK = 8000
N_E_CHUNKS = E_CG // E_CHUNK
A_PER_W = N_ATOM_PAD // NW
A_GATHER_ROWS = 128
A_N_GATHERS = A_PER_W // A_GATHER_ROWS
CA_PAD = 10240


def _gate_body(cz_ref, embT_ref, w1T_ref, wgT_ref, out_ref):
    cz = cz_ref[0, 0, :]
    rows = lax.broadcasted_iota(jnp.int32, (D, BN), 0)
    oh = (rows == cz[None, :]).astype(jnp.float32)
    st = jnp.dot(embT_ref[...], oh, preferred_element_type=jnp.float32)
    ht = jnp.maximum(
        jnp.dot(w1T_ref[...], st, preferred_element_type=jnp.float32), 0.0)
    out_ref[...] = jnp.dot(wgT_ref[...], ht,
                           preferred_element_type=jnp.float32)


def _gate_features(cz_pad, embT_pad, w1T, wgT):
    n_blocks = N_CG_PAD // BN
    czp = cz_pad.reshape(n_blocks, 1, BN)
    return pl.pallas_call(
        _gate_body,
        grid=(n_blocks,),
        in_specs=[
            pl.BlockSpec((1, 1, BN), lambda i: (i, 0, 0)),
            pl.BlockSpec((D, D), lambda i: (0, 0)),
            pl.BlockSpec((D, D), lambda i: (0, 0)),
            pl.BlockSpec((C, D), lambda i: (0, 0)),
        ],
        out_specs=pl.BlockSpec((C, BN), lambda i: (0, i)),
        out_shape=jax.ShapeDtypeStruct((C, N_CG_PAD), jnp.float32),
    )(czp, embT_pad, w1T, wgT)


def _fast_rsqrt(x):
    i = plsc.bitcast(x, jnp.int32)
    y = plsc.bitcast(jnp.int32(0x5F3759DF) - (i >> 1), jnp.float32)
    h = x * 0.5
    y = y * (1.5 - h * y * y)
    y = y * (1.5 - h * y * y)
    y = y * (1.5 - h * y * y)
    return y


def _wid():
    return lax.axis_index("s") * NC + lax.axis_index("c")


@functools.cache
def _build_edge_kernel():
    mesh = plsc.VectorSubcoreMesh(core_axis_name="c", subcore_axis_name="s")
    return functools.partial(
        pl.kernel,
        mesh=mesh,
        out_type=jax.ShapeDtypeStruct((NW, N_CG_PAD * 4), jnp.float32),
        scratch_types=[
            pltpu.VMEM((3 * N_CG_PAD,), jnp.float32),
            pltpu.VMEM((N_CG_PAD,), jnp.float32),
            pltpu.VMEM((N_CG_PAD * 4,), jnp.float32),
            pltpu.VMEM((E_CHUNK,), jnp.int32),
            pltpu.VMEM((E_CHUNK,), jnp.int32),
        ],
        compiler_params=pltpu.CompilerParams(
            needs_layout_passes=False, use_tc_tiling_on_sc=False),
    )(_edge_body)


def _edge_body(xyzT_hbm, gT_hbm, src_hbm, dst_hbm, v_hbm,
               xyz_v, g_v, acc_v, src_v, dst_v):
    w = _wid()
    pltpu.sync_copy(xyzT_hbm, xyz_v)
    pltpu.sync_copy(gT_hbm.at[w], g_v)

    zeros16 = jnp.zeros((L,), jnp.float32)

    def zero_body(i, carry):
        acc_v[pl.ds(i * L, L)] = zeros16
        return carry

    lax.fori_loop(0, N_CG_PAD * 4 // L, zero_body, 0)

    o1 = jnp.full((L,), N_CG_PAD, jnp.int32)
    o2 = jnp.full((L,), 2 * N_CG_PAD, jnp.int32)
    one16 = jnp.full((L,), 1, jnp.int32)
    two16 = jnp.full((L,), 2, jnp.int32)

    def chunk_body(g, carry):
        base = g * E_CHUNK
        pltpu.sync_copy(src_hbm.at[pl.ds(base, E_CHUNK)], src_v)
        pltpu.sync_copy(dst_hbm.at[pl.ds(base, E_CHUNK)], dst_v)

        def grp_body(i, c2):
            sl = pl.ds(i * L, L)
            s = src_v[sl]
            d = dst_v[sl]
            xs = plsc.load_gather(xyz_v, [s])
            ys = plsc.load_gather(xyz_v, [s + o1])
            zs = plsc.load_gather(xyz_v, [s + o2])
            xd = plsc.load_gather(xyz_v, [d])
            yd = plsc.load_gather(xyz_v, [d + o1])
            zd = plsc.load_gather(xyz_v, [d + o2])
            rx = xd - xs
            ry = yd - ys
            rz = zd - zs
            s2 = rx * rx + ry * ry + rz * rz
            rs = _fast_rsqrt(jnp.maximum(s2, 1e-30))
            dn = s2 * rs
            inv = 1.0 / (dn + 1e-8)
            gv = plsc.load_gather(g_v, [s])
            wv = gv * inv
            d4 = d * 4
            plsc.addupdate_scatter(acc_v, [d4], wv * rx)
            plsc.addupdate_scatter(acc_v, [d4 + one16], wv * ry)
            plsc.addupdate_scatter(acc_v, [d4 + two16], wv * rz)
            return c2

        lax.fori_loop(0, E_CHUNK // L, grp_body, 0)
        return carry

    lax.fori_loop(0, N_E_CHUNKS, chunk_body, 0)
    pltpu.sync_copy(acc_v, v_hbm.at[w])


@functools.cache
def _build_atom_kernel():
    mesh = plsc.VectorSubcoreMesh(core_axis_name="c", subcore_axis_name="s")
    return functools.partial(
        pl.kernel,
        mesh=mesh,
        out_type=jax.ShapeDtypeStruct((NW, 3, A_PER_W), jnp.float32),
        scratch_types=[
            pltpu.VMEM((3 * N_CG_PAD,), jnp.float32),
            pltpu.VMEM((A_PER_W + 32,), jnp.int32),
            pltpu.VMEM((A_PER_W,), jnp.int32),
            pltpu.VMEM((A_PER_W,), jnp.int32),
            pltpu.VMEM((A_PER_W,), jnp.int32),
            pltpu.VMEM((A_PER_W,), jnp.float32),
            pltpu.VMEM((A_PER_W,), jnp.float32),
            pltpu.VMEM((A_PER_W,), jnp.float32),
            pltpu.VMEM((A_PER_W,), jnp.float32),
            pltpu.VMEM((A_PER_W,), jnp.float32),
            pltpu.VMEM((A_PER_W,), jnp.float32),
            pltpu.VMEM((CA_PAD,), jnp.int32),
            pltpu.SemaphoreType.DMA,
        ],
        compiler_params=pltpu.CompilerParams(
            needs_layout_passes=False, use_tc_tiling_on_sc=False),
    )(_atom_body)


def _atom_body(xyzT_hbm, mext_hbm, v4_hbm, ca_hbm, out_hbm,
               xyz_v, m_v, rix_v, riy_v, riz_v, vrx_v, vry_v, vrz_v,
               ox_v, oy_v, oz_v, ca_v, sem):
    w = _wid()
    abase = w * A_PER_W
    pltpu.sync_copy(xyzT_hbm, xyz_v)
    pltpu.sync_copy(mext_hbm.at[pl.ds(abase, A_PER_W + 32)], m_v)
    pltpu.sync_copy(ca_hbm, ca_v)

    iota = lax.iota(jnp.int32, L)
    o1 = jnp.full((L,), N_CG_PAD, jnp.int32)
    o2 = jnp.full((L,), 2 * N_CG_PAD, jnp.int32)
    one16 = jnp.full((L,), 1, jnp.int32)
    two16 = jnp.full((L,), 2, jnp.int32)

    def pass1(i, carry):
        q = i * L
        mcur = m_v[pl.ds(32 + q, L)]
        ch = jnp.zeros((L,), jnp.int32)
        for j in range(1, 32):
            mj = plsc.load_gather(m_v, [iota + (32 + q - j)])
            ch = ch + (mj == mcur).astype(jnp.int32)
        r4 = (ch * N_CG_PAD + mcur) * 4
        sl = pl.ds(q, L)
        rix_v[sl] = r4
        riy_v[sl] = r4 + one16
        riz_v[sl] = r4 + two16
        return carry

    lax.fori_loop(0, A_PER_W // L, pass1, 0)

    copies = []
    for ridx, vdst in ((rix_v, vrx_v), (riy_v, vry_v), (riz_v, vrz_v)):
        for j in range(A_N_GATHERS):
            sl = pl.ds(j * A_GATHER_ROWS, A_GATHER_ROWS)
            copies.append(
                pltpu.async_copy(v4_hbm.at[ridx.at[sl]], vdst.at[sl], sem))
    for cp in copies:
        cp.wait()

    def pass2(i, carry):
        q = i * L
        sl = pl.ds(q, L)
        mcur = m_v[pl.ds(32 + q, L)]
        bx = plsc.load_gather(xyz_v, [mcur])
        by = plsc.load_gather(xyz_v, [mcur + o1])
        bz = plsc.load_gather(xyz_v, [mcur + o2])
        ox_v[sl] = vrx_v[sl] + bx
        oy_v[sl] = vry_v[sl] + by
        oz_v[sl] = vrz_v[sl] + bz
        return carry

    lax.fori_loop(0, A_PER_W // L, pass2, 0)

    def pass3(i, carry):
        t = ca_v[pl.ds(i * L, L)]
        lt = t - abase
        inb = (lt >= 0) & (lt < A_PER_W)
        lsafe = jnp.where(inb, lt, 0)
        mt = plsc.load_gather(m_v, [lsafe + 32])
        bx = plsc.load_gather(xyz_v, [mt])
        by = plsc.load_gather(xyz_v, [mt + o1])
        bz = plsc.load_gather(xyz_v, [mt + o2])
        plsc.store_scatter(ox_v, [lsafe], bx, mask=inb)
        plsc.store_scatter(oy_v, [lsafe], by, mask=inb)
        plsc.store_scatter(oz_v, [lsafe], bz, mask=inb)
        return carry

    lax.fori_loop(0, CA_PAD // L, pass3, 0)

    pltpu.sync_copy(ox_v, out_hbm.at[w, 0])
    pltpu.sync_copy(oy_v, out_hbm.at[w, 1])
    pltpu.sync_copy(oz_v, out_hbm.at[w, 2])


def kernel(cg_z, xyz, cg_xyz, bond_edge_list, CG_nbr_list, cg_map, ca_idx,
           emb, W1, Wg):
    f32 = jnp.float32
    i32 = jnp.int32

    cz_pad = jnp.zeros((N_CG_PAD,), i32).at[:N_CG].set(cg_z.astype(i32))
    embT_pad = jnp.zeros((D, D), f32).at[:, :emb.shape[0]].set(emb.T)
    gT = _gate_features(cz_pad, embT_pad, W1.T, Wg.T)

    xyzT = jnp.zeros((3, N_CG_PAD), f32).at[:, :N_CG].set(cg_xyz.T)
    xyzT_flat = xyzT.reshape(3 * N_CG_PAD)
    src = CG_nbr_list[:, 0].astype(i32)
    dst = CG_nbr_list[:, 1].astype(i32)

    v_flat = _build_edge_kernel()(xyzT_flat, gT, src, dst)
    v4 = v_flat.reshape(NW * N_CG_PAD * 4)

    mext = jnp.concatenate([
        jnp.full((32,), -1, i32),
        cg_map.astype(i32),
        jnp.zeros((N_ATOM_PAD - N_ATOM,), i32),
    ])
    ca_pad = jnp.concatenate([
        ca_idx.astype(i32),
        jnp.full((CA_PAD - ca_idx.shape[0],), -1, i32),
    ])

    out = _build_atom_kernel()(xyzT_flat, mext, v4, ca_pad)
    xyz_recon = out.transpose(0, 2, 1).reshape(N_ATOM_PAD, 3)[:N_ATOM]
    return (xyz, xyz_recon)


# speedup vs baseline: 32.5704x; 1.0830x over previous
import functools

import jax
import jax.numpy as jnp
from jax import lax
from jax.experimental import pallas as pl
from jax.experimental.pallas import tpu as pltpu
from jax.experimental.pallas import tpu_sc as plsc

N_CG = 10000
N_CG_PAD = 10240
N_ATOM = 80000
N_ATOM_PAD = 81920
E_CG = 320000
D = 128
C = 32

NC = 2
NS = 16
NW = NC * NS
L = 16

BN = 1280
E_CHUNK = 8000
N_E_CHUNKS = E_CG // E_CHUNK
A_PER_W = N_ATOM_PAD // NW
A_GATHER_ROWS = 128
A_N_GATHERS = A_PER_W // A_GATHER_ROWS
CA_PAD = 10240


def _gate_body(cz_ref, embT_ref, w1T_ref, wgT_ref, out_ref):
    cz = cz_ref[0, 0, :]
    rows = lax.broadcasted_iota(jnp.int32, (D, BN), 0)
    oh = (rows == cz[None, :]).astype(jnp.float32)
    st = jnp.dot(embT_ref[...], oh, preferred_element_type=jnp.float32)
    ht = jnp.maximum(
        jnp.dot(w1T_ref[...], st, preferred_element_type=jnp.float32), 0.0)
    out_ref[...] = jnp.dot(wgT_ref[...], ht,
                           preferred_element_type=jnp.float32)


def _gate_features(cz_pad, embT_pad, w1T, wgT):
    n_blocks = N_CG_PAD // BN
    czp = cz_pad.reshape(n_blocks, 1, BN)
    return pl.pallas_call(
        _gate_body,
        grid=(n_blocks,),
        in_specs=[
            pl.BlockSpec((1, 1, BN), lambda i: (i, 0, 0)),
            pl.BlockSpec((D, D), lambda i: (0, 0)),
            pl.BlockSpec((D, D), lambda i: (0, 0)),
            pl.BlockSpec((C, D), lambda i: (0, 0)),
        ],
        out_specs=pl.BlockSpec((C, BN), lambda i: (0, i)),
        out_shape=jax.ShapeDtypeStruct((C, N_CG_PAD), jnp.float32),
    )(czp, embT_pad, w1T, wgT)


def _fast_rsqrt(x):
    i = plsc.bitcast(x, jnp.int32)
    y = plsc.bitcast(jnp.int32(0x5F3759DF) - (i >> 1), jnp.float32)
    h = x * 0.5
    y = y * (1.5 - h * y * y)
    y = y * (1.5 - h * y * y)
    y = y * (1.5 - h * y * y)
    return y


def _wid():
    return lax.axis_index("s") * NC + lax.axis_index("c")


@functools.cache
def _build_edge_kernel():
    mesh = plsc.VectorSubcoreMesh(core_axis_name="c", subcore_axis_name="s")
    return functools.partial(
        pl.kernel,
        mesh=mesh,
        out_type=jax.ShapeDtypeStruct((NW, 3 * N_CG_PAD), jnp.float32),
        scratch_types=[
            pltpu.VMEM((3 * N_CG_PAD,), jnp.float32),
            pltpu.VMEM((N_CG_PAD,), jnp.float32),
            pltpu.VMEM((3 * N_CG_PAD,), jnp.float32),
            pltpu.VMEM((E_CHUNK,), jnp.int32),
            pltpu.VMEM((E_CHUNK,), jnp.int32),
        ],
        compiler_params=pltpu.CompilerParams(
            needs_layout_passes=False, use_tc_tiling_on_sc=False),
    )(_edge_body)


def _edge_body(xyzT_hbm, gT_hbm, src_hbm, dst_hbm, v_hbm,
               xyz_v, g_v, acc_v, src_v, dst_v):
    w = _wid()
    pltpu.sync_copy(xyzT_hbm, xyz_v)
    pltpu.sync_copy(gT_hbm.at[w], g_v)

    zeros16 = jnp.zeros((L,), jnp.float32)

    def zero_body(i, carry):
        acc_v[pl.ds(i * L, L)] = zeros16
        return carry

    lax.fori_loop(0, 3 * N_CG_PAD // L, zero_body, 0)

    o1 = jnp.full((L,), N_CG_PAD, jnp.int32)
    o2 = jnp.full((L,), 2 * N_CG_PAD, jnp.int32)

    def chunk_body(g, carry):
        base = g * E_CHUNK
        pltpu.sync_copy(src_hbm.at[pl.ds(base, E_CHUNK)], src_v)
        pltpu.sync_copy(dst_hbm.at[pl.ds(base, E_CHUNK)], dst_v)

        def grp_body(i, c2):
            sl = pl.ds(i * L, L)
            s = src_v[sl]
            d = dst_v[sl]
            xs = plsc.load_gather(xyz_v, [s])
            ys = plsc.load_gather(xyz_v, [s + o1])
            zs = plsc.load_gather(xyz_v, [s + o2])
            xd = plsc.load_gather(xyz_v, [d])
            yd = plsc.load_gather(xyz_v, [d + o1])
            zd = plsc.load_gather(xyz_v, [d + o2])
            rx = xd - xs
            ry = yd - ys
            rz = zd - zs
            s2 = rx * rx + ry * ry + rz * rz
            rs = _fast_rsqrt(jnp.maximum(s2, 1e-30))
            dn = s2 * rs
            inv = 1.0 / (dn + 1e-8)
            gv = plsc.load_gather(g_v, [s])
            wv = gv * inv
            plsc.addupdate_scatter(acc_v, [d], wv * rx)
            plsc.addupdate_scatter(acc_v, [d + o1], wv * ry)
            plsc.addupdate_scatter(acc_v, [d + o2], wv * rz)
            return c2

        lax.fori_loop(0, E_CHUNK // L, grp_body, 0)
        return carry

    lax.fori_loop(0, N_E_CHUNKS, chunk_body, 0)
    pltpu.sync_copy(acc_v, v_hbm.at[w])


@functools.cache
def _build_atom_kernel():
    mesh = plsc.VectorSubcoreMesh(core_axis_name="c", subcore_axis_name="s")
    return functools.partial(
        pl.kernel,
        mesh=mesh,
        out_type=jax.ShapeDtypeStruct((NW, 3, A_PER_W), jnp.float32),
        scratch_types=[
            pltpu.VMEM((3 * N_CG_PAD,), jnp.float32),
            pltpu.VMEM((A_PER_W + 32,), jnp.int32),
            pltpu.VMEM((A_PER_W,), jnp.int32),
            pltpu.VMEM((A_PER_W,), jnp.int32),
            pltpu.VMEM((A_PER_W,), jnp.int32),
            pltpu.VMEM((A_PER_W,), jnp.float32),
            pltpu.VMEM((A_PER_W,), jnp.float32),
            pltpu.VMEM((A_PER_W,), jnp.float32),
            pltpu.VMEM((A_PER_W,), jnp.float32),
            pltpu.VMEM((A_PER_W,), jnp.float32),
            pltpu.VMEM((A_PER_W,), jnp.float32),
            pltpu.VMEM((CA_PAD,), jnp.int32),
            pltpu.SemaphoreType.DMA,
        ],
        compiler_params=pltpu.CompilerParams(
            needs_layout_passes=False, use_tc_tiling_on_sc=False),
    )(_atom_body)


def _atom_body(xyzT_hbm, mext_hbm, v4_hbm, ca_hbm, out_hbm,
               xyz_v, m_v, rix_v, riy_v, riz_v, vrx_v, vry_v, vrz_v,
               ox_v, oy_v, oz_v, ca_v, sem):
    w = _wid()
    abase = w * A_PER_W
    pltpu.sync_copy(xyzT_hbm, xyz_v)
    pltpu.sync_copy(mext_hbm.at[pl.ds(abase, A_PER_W + 32)], m_v)
    pltpu.sync_copy(ca_hbm, ca_v)

    iota = lax.iota(jnp.int32, L)
    o1 = jnp.full((L,), N_CG_PAD, jnp.int32)
    o2 = jnp.full((L,), 2 * N_CG_PAD, jnp.int32)

    def pass1(i, carry):
        q = i * L
        mcur = m_v[pl.ds(32 + q, L)]
        ch = jnp.zeros((L,), jnp.int32)
        for j in range(1, 32):
            mj = plsc.load_gather(m_v, [iota + (32 + q - j)])
            ch = ch + (mj == mcur).astype(jnp.int32)
        r0 = ch * (3 * N_CG_PAD) + mcur
        sl = pl.ds(q, L)
        rix_v[sl] = r0
        riy_v[sl] = r0 + o1
        riz_v[sl] = r0 + o2
        return carry

    lax.fori_loop(0, A_PER_W // L, pass1, 0)

    copies = []
    for ridx, vdst in ((rix_v, vrx_v), (riy_v, vry_v), (riz_v, vrz_v)):
        for j in range(A_N_GATHERS):
            sl = pl.ds(j * A_GATHER_ROWS, A_GATHER_ROWS)
            copies.append(
                pltpu.async_copy(v4_hbm.at[ridx.at[sl]], vdst.at[sl], sem))
    for cp in copies:
        cp.wait()

    def pass2(i, carry):
        q = i * L
        sl = pl.ds(q, L)
        mcur = m_v[pl.ds(32 + q, L)]
        bx = plsc.load_gather(xyz_v, [mcur])
        by = plsc.load_gather(xyz_v, [mcur + o1])
        bz = plsc.load_gather(xyz_v, [mcur + o2])
        ox_v[sl] = vrx_v[sl] + bx
        oy_v[sl] = vry_v[sl] + by
        oz_v[sl] = vrz_v[sl] + bz
        return carry

    lax.fori_loop(0, A_PER_W // L, pass2, 0)

    def pass3(i, carry):
        t = ca_v[pl.ds(i * L, L)]
        lt = t - abase
        inb = (lt >= 0) & (lt < A_PER_W)
        lsafe = jnp.where(inb, lt, 0)
        mt = plsc.load_gather(m_v, [lsafe + 32])
        bx = plsc.load_gather(xyz_v, [mt])
        by = plsc.load_gather(xyz_v, [mt + o1])
        bz = plsc.load_gather(xyz_v, [mt + o2])
        plsc.store_scatter(ox_v, [lsafe], bx, mask=inb)
        plsc.store_scatter(oy_v, [lsafe], by, mask=inb)
        plsc.store_scatter(oz_v, [lsafe], bz, mask=inb)
        return carry

    lax.fori_loop(0, CA_PAD // L, pass3, 0)

    pltpu.sync_copy(ox_v, out_hbm.at[w, 0])
    pltpu.sync_copy(oy_v, out_hbm.at[w, 1])
    pltpu.sync_copy(oz_v, out_hbm.at[w, 2])


def kernel(cg_z, xyz, cg_xyz, bond_edge_list, CG_nbr_list, cg_map, ca_idx,
           emb, W1, Wg):
    f32 = jnp.float32
    i32 = jnp.int32

    cz_pad = jnp.zeros((N_CG_PAD,), i32).at[:N_CG].set(cg_z.astype(i32))
    embT_pad = jnp.zeros((D, D), f32).at[:, :emb.shape[0]].set(emb.T)
    gT = _gate_features(cz_pad, embT_pad, W1.T, Wg.T)

    xyzT = jnp.zeros((3, N_CG_PAD), f32).at[:, :N_CG].set(cg_xyz.T)
    xyzT_flat = xyzT.reshape(3 * N_CG_PAD)
    src = CG_nbr_list[:, 0].astype(i32)
    dst = CG_nbr_list[:, 1].astype(i32)

    v_flat = _build_edge_kernel()(xyzT_flat, gT, src, dst)
    v4 = v_flat.reshape(NW * 3 * N_CG_PAD)

    mext = jnp.concatenate([
        jnp.full((32,), -1, i32),
        cg_map.astype(i32),
        jnp.zeros((N_ATOM_PAD - N_ATOM,), i32),
    ])
    ca_pad = jnp.concatenate([
        ca_idx.astype(i32),
        jnp.full((CA_PAD - ca_idx.shape[0],), -1, i32),
    ])

    out = _build_atom_kernel()(xyzT_flat, mext, v4, ca_pad)
    xyz_recon = out.transpose(0, 2, 1).reshape(N_ATOM_PAD, 3)[:N_ATOM]
    return (xyz, xyz_recon)


# speedup vs baseline: 71.6858x; 2.2010x over previous
import functools

import jax
import jax.numpy as jnp
from jax import lax
from jax.experimental import pallas as pl
from jax.experimental.pallas import tpu as pltpu
from jax.experimental.pallas import tpu_sc as plsc

N_CG = 10000
N_CG_PAD = 10240
N_ATOM = 80000
N_ATOM_PAD = 81920
E_CG = 320000
D = 128
C = 32

NC = 2
NS = 16
NW = NC * NS
L = 16

BN = 1280
E_CHUNK = 8000
N_E_CHUNKS = E_CG // E_CHUNK
A_PER_W = N_ATOM_PAD // NW
A_GATHER_ROWS = 128
A_N_GATHERS = A_PER_W // A_GATHER_ROWS
CA_PAD = 10240


def _gate_body(cz_ref, embT_ref, w1T_ref, wgT_ref, out_ref):
    cz = cz_ref[0, 0, :]
    rows = lax.broadcasted_iota(jnp.int32, (D, BN), 0)
    oh = (rows == cz[None, :]).astype(jnp.float32)
    st = jnp.dot(embT_ref[...], oh, preferred_element_type=jnp.float32)
    ht = jnp.maximum(
        jnp.dot(w1T_ref[...], st, preferred_element_type=jnp.float32), 0.0)
    out_ref[...] = jnp.dot(wgT_ref[...], ht,
                           preferred_element_type=jnp.float32)


def _gate_features(cz_pad, embT_pad, w1T, wgT):
    n_blocks = N_CG_PAD // BN
    czp = cz_pad.reshape(n_blocks, 1, BN)
    return pl.pallas_call(
        _gate_body,
        grid=(n_blocks,),
        in_specs=[
            pl.BlockSpec((1, 1, BN), lambda i: (i, 0, 0)),
            pl.BlockSpec((D, D), lambda i: (0, 0)),
            pl.BlockSpec((D, D), lambda i: (0, 0)),
            pl.BlockSpec((C, D), lambda i: (0, 0)),
        ],
        out_specs=pl.BlockSpec((C, BN), lambda i: (0, i)),
        out_shape=jax.ShapeDtypeStruct((C, N_CG_PAD), jnp.float32),
    )(czp, embT_pad, w1T, wgT)


def _fast_rsqrt(x):
    i = plsc.bitcast(x, jnp.int32)
    y = plsc.bitcast(jnp.int32(0x5F3759DF) - (i >> 1), jnp.float32)
    h = x * 0.5
    y = y * (1.5 - h * y * y)
    y = y * (1.5 - h * y * y)
    y = y * (1.5 - h * y * y)
    return y


def _wid():
    return lax.axis_index("s") * NC + lax.axis_index("c")


@functools.cache
def _build_edge_kernel():
    mesh = plsc.VectorSubcoreMesh(core_axis_name="c", subcore_axis_name="s")
    return functools.partial(
        pl.kernel,
        mesh=mesh,
        out_type=jax.ShapeDtypeStruct((NW, 3 * N_CG_PAD), jnp.float32),
        scratch_types=[
            pltpu.VMEM((3 * N_CG_PAD,), jnp.float32),
            pltpu.VMEM((N_CG_PAD,), jnp.float32),
            pltpu.VMEM((3 * N_CG_PAD,), jnp.float32),
            pltpu.VMEM((E_CHUNK,), jnp.int32),
            pltpu.VMEM((E_CHUNK,), jnp.int32),
        ],
        compiler_params=pltpu.CompilerParams(
            needs_layout_passes=False, use_tc_tiling_on_sc=False),
    )(_edge_body)


def _edge_body(xyzT_hbm, gT_hbm, src_hbm, dst_hbm, v_hbm,
               xyz_v, g_v, acc_v, src_v, dst_v):
    w = _wid()
    pltpu.sync_copy(xyzT_hbm, xyz_v)
    pltpu.sync_copy(gT_hbm.at[w], g_v)

    zeros16 = jnp.zeros((L,), jnp.float32)

    @plsc.parallel_loop(0, 3 * N_CG_PAD // L, unroll=8)
    def zero_body(i):
        acc_v[pl.ds(i * L, L)] = zeros16

    o1 = jnp.full((L,), N_CG_PAD, jnp.int32)
    o2 = jnp.full((L,), 2 * N_CG_PAD, jnp.int32)

    def chunk_body(g, carry):
        base = g * E_CHUNK
        pltpu.sync_copy(src_hbm.at[pl.ds(base, E_CHUNK)], src_v)
        pltpu.sync_copy(dst_hbm.at[pl.ds(base, E_CHUNK)], dst_v)

        @plsc.parallel_loop(0, E_CHUNK // L, unroll=4)
        def grp_body(i):
            sl = pl.ds(i * L, L)
            s = src_v[sl]
            d = dst_v[sl]
            xs = plsc.load_gather(xyz_v, [s])
            ys = plsc.load_gather(xyz_v, [s + o1])
            zs = plsc.load_gather(xyz_v, [s + o2])
            xd = plsc.load_gather(xyz_v, [d])
            yd = plsc.load_gather(xyz_v, [d + o1])
            zd = plsc.load_gather(xyz_v, [d + o2])
            rx = xd - xs
            ry = yd - ys
            rz = zd - zs
            s2 = rx * rx + ry * ry + rz * rz
            rs = _fast_rsqrt(jnp.maximum(s2, 1e-30))
            dn = s2 * rs
            inv = 1.0 / (dn + 1e-8)
            gv = plsc.load_gather(g_v, [s])
            wv = gv * inv
            plsc.addupdate_scatter(acc_v, [d], wv * rx)
            plsc.addupdate_scatter(acc_v, [d + o1], wv * ry)
            plsc.addupdate_scatter(acc_v, [d + o2], wv * rz)
        return carry

    lax.fori_loop(0, N_E_CHUNKS, chunk_body, 0)
    pltpu.sync_copy(acc_v, v_hbm.at[w])


@functools.cache
def _build_atom_kernel():
    mesh = plsc.VectorSubcoreMesh(core_axis_name="c", subcore_axis_name="s")
    return functools.partial(
        pl.kernel,
        mesh=mesh,
        out_type=jax.ShapeDtypeStruct((NW, 3, A_PER_W), jnp.float32),
        scratch_types=[
            pltpu.VMEM((3 * N_CG_PAD,), jnp.float32),
            pltpu.VMEM((A_PER_W + 32,), jnp.int32),
            pltpu.VMEM((A_PER_W,), jnp.int32),
            pltpu.VMEM((A_PER_W,), jnp.int32),
            pltpu.VMEM((A_PER_W,), jnp.int32),
            pltpu.VMEM((A_PER_W,), jnp.float32),
            pltpu.VMEM((A_PER_W,), jnp.float32),
            pltpu.VMEM((A_PER_W,), jnp.float32),
            pltpu.VMEM((A_PER_W,), jnp.float32),
            pltpu.VMEM((A_PER_W,), jnp.float32),
            pltpu.VMEM((A_PER_W,), jnp.float32),
            pltpu.VMEM((CA_PAD,), jnp.int32),
            pltpu.SemaphoreType.DMA,
        ],
        compiler_params=pltpu.CompilerParams(
            needs_layout_passes=False, use_tc_tiling_on_sc=False),
    )(_atom_body)


def _atom_body(xyzT_hbm, mext_hbm, v4_hbm, ca_hbm, out_hbm,
               xyz_v, m_v, rix_v, riy_v, riz_v, vrx_v, vry_v, vrz_v,
               ox_v, oy_v, oz_v, ca_v, sem):
    w = _wid()
    abase = w * A_PER_W
    pltpu.sync_copy(xyzT_hbm, xyz_v)
    pltpu.sync_copy(mext_hbm.at[pl.ds(abase, A_PER_W + 32)], m_v)
    pltpu.sync_copy(ca_hbm, ca_v)

    iota = lax.iota(jnp.int32, L)
    o1 = jnp.full((L,), N_CG_PAD, jnp.int32)
    o2 = jnp.full((L,), 2 * N_CG_PAD, jnp.int32)

    @plsc.parallel_loop(0, A_PER_W // L, unroll=2)
    def pass1(i):
        q = i * L
        mcur = m_v[pl.ds(32 + q, L)]
        ch = jnp.zeros((L,), jnp.int32)
        for j in range(1, 32):
            mj = plsc.load_gather(m_v, [iota + (32 + q - j)])
            ch = ch + (mj == mcur).astype(jnp.int32)
        r0 = ch * (3 * N_CG_PAD) + mcur
        sl = pl.ds(q, L)
        rix_v[sl] = r0
        riy_v[sl] = r0 + o1
        riz_v[sl] = r0 + o2

    copies = []
    for ridx, vdst in ((rix_v, vrx_v), (riy_v, vry_v), (riz_v, vrz_v)):
        for j in range(A_N_GATHERS):
            sl = pl.ds(j * A_GATHER_ROWS, A_GATHER_ROWS)
            copies.append(
                pltpu.async_copy(v4_hbm.at[ridx.at[sl]], vdst.at[sl], sem))
    for cp in copies:
        cp.wait()

    @plsc.parallel_loop(0, A_PER_W // L, unroll=4)
    def pass2(i):
        q = i * L
        sl = pl.ds(q, L)
        mcur = m_v[pl.ds(32 + q, L)]
        bx = plsc.load_gather(xyz_v, [mcur])
        by = plsc.load_gather(xyz_v, [mcur + o1])
        bz = plsc.load_gather(xyz_v, [mcur + o2])
        ox_v[sl] = vrx_v[sl] + bx
        oy_v[sl] = vry_v[sl] + by
        oz_v[sl] = vrz_v[sl] + bz

    @plsc.parallel_loop(0, CA_PAD // L, unroll=4)
    def pass3(i):
        t = ca_v[pl.ds(i * L, L)]
        lt = t - abase
        inb = (lt >= 0) & (lt < A_PER_W)
        lsafe = jnp.where(inb, lt, 0)
        mt = plsc.load_gather(m_v, [lsafe + 32])
        bx = plsc.load_gather(xyz_v, [mt])
        by = plsc.load_gather(xyz_v, [mt + o1])
        bz = plsc.load_gather(xyz_v, [mt + o2])
        plsc.store_scatter(ox_v, [lsafe], bx, mask=inb)
        plsc.store_scatter(oy_v, [lsafe], by, mask=inb)
        plsc.store_scatter(oz_v, [lsafe], bz, mask=inb)

    pltpu.sync_copy(ox_v, out_hbm.at[w, 0])
    pltpu.sync_copy(oy_v, out_hbm.at[w, 1])
    pltpu.sync_copy(oz_v, out_hbm.at[w, 2])


def kernel(cg_z, xyz, cg_xyz, bond_edge_list, CG_nbr_list, cg_map, ca_idx,
           emb, W1, Wg):
    f32 = jnp.float32
    i32 = jnp.int32

    cz_pad = jnp.zeros((N_CG_PAD,), i32).at[:N_CG].set(cg_z.astype(i32))
    embT_pad = jnp.zeros((D, D), f32).at[:, :emb.shape[0]].set(emb.T)
    gT = _gate_features(cz_pad, embT_pad, W1.T, Wg.T)

    xyzT = jnp.zeros((3, N_CG_PAD), f32).at[:, :N_CG].set(cg_xyz.T)
    xyzT_flat = xyzT.reshape(3 * N_CG_PAD)
    src = CG_nbr_list[:, 0].astype(i32)
    dst = CG_nbr_list[:, 1].astype(i32)

    v_flat = _build_edge_kernel()(xyzT_flat, gT, src, dst)
    v4 = v_flat.reshape(NW * 3 * N_CG_PAD)

    mext = jnp.concatenate([
        jnp.full((32,), -1, i32),
        cg_map.astype(i32),
        jnp.zeros((N_ATOM_PAD - N_ATOM,), i32),
    ])
    ca_pad = jnp.concatenate([
        ca_idx.astype(i32),
        jnp.full((CA_PAD - ca_idx.shape[0],), -1, i32),
    ])

    out = _build_atom_kernel()(xyzT_flat, mext, v4, ca_pad)
    xyz_recon = out.transpose(0, 2, 1).reshape(N_ATOM_PAD, 3)[:N_ATOM]
    return (xyz, xyz_recon)


# speedup vs baseline: 138.4982x; 1.9320x over previous
import functools

import jax
import jax.numpy as jnp
from jax import lax
from jax.experimental import pallas as pl
from jax.experimental.pallas import tpu as pltpu
from jax.experimental.pallas import tpu_sc as plsc

N_CG = 10000
N_CG_PAD = 10240
N_ATOM = 80000
N_ATOM_PAD = 81920
E_CG = 320000
D = 128
C = 32

NC = 2
NS = 16
NW = NC * NS
L = 16

BN = 1280
E_CHUNK = 6400
N_E_CHUNKS = E_CG // E_CHUNK
A_PER_W = N_ATOM_PAD // NW
A_GATHER_ROWS = 128
A_N_GATHERS = A_PER_W // A_GATHER_ROWS
CA_PAD = 10240


def _gate_body(cz_ref, embT_ref, w1T_ref, wgT_ref, out_ref):
    cz = cz_ref[0, 0, :]
    rows = lax.broadcasted_iota(jnp.int32, (D, BN), 0)
    oh = (rows == cz[None, :]).astype(jnp.float32)
    st = jnp.dot(embT_ref[...], oh, preferred_element_type=jnp.float32)
    ht = jnp.maximum(
        jnp.dot(w1T_ref[...], st, preferred_element_type=jnp.float32), 0.0)
    out_ref[...] = jnp.dot(wgT_ref[...], ht,
                           preferred_element_type=jnp.float32)


def _gate_features(cz_pad, embT_pad, w1T, wgT):
    n_blocks = N_CG_PAD // BN
    czp = cz_pad.reshape(n_blocks, 1, BN)
    return pl.pallas_call(
        _gate_body,
        grid=(n_blocks,),
        in_specs=[
            pl.BlockSpec((1, 1, BN), lambda i: (i, 0, 0)),
            pl.BlockSpec((D, D), lambda i: (0, 0)),
            pl.BlockSpec((D, D), lambda i: (0, 0)),
            pl.BlockSpec((C, D), lambda i: (0, 0)),
        ],
        out_specs=pl.BlockSpec((C, BN), lambda i: (0, i)),
        out_shape=jax.ShapeDtypeStruct((C, N_CG_PAD), jnp.float32),
    )(czp, embT_pad, w1T, wgT)


def _fast_rsqrt(x):
    i = plsc.bitcast(x, jnp.int32)
    y = plsc.bitcast(jnp.int32(0x5F3759DF) - (i >> 1), jnp.float32)
    h = x * 0.5
    y = y * (1.5 - h * y * y)
    y = y * (1.5 - h * y * y)
    y = y * (1.5 - h * y * y)
    return y


def _wid():
    return lax.axis_index("s") * NC + lax.axis_index("c")


E_SLICE = E_CG // NW


@functools.cache
def _build_unit_kernel():
    mesh = plsc.VectorSubcoreMesh(core_axis_name="c", subcore_axis_name="s")
    return functools.partial(
        pl.kernel,
        mesh=mesh,
        out_type=jax.ShapeDtypeStruct((3, E_CG), jnp.float32),
        scratch_types=[
            pltpu.VMEM((3 * N_CG_PAD,), jnp.float32),
            pltpu.VMEM((E_SLICE,), jnp.int32),
            pltpu.VMEM((E_SLICE,), jnp.int32),
            pltpu.VMEM((E_SLICE,), jnp.float32),
            pltpu.VMEM((E_SLICE,), jnp.float32),
            pltpu.VMEM((E_SLICE,), jnp.float32),
        ],
        compiler_params=pltpu.CompilerParams(
            needs_layout_passes=False, use_tc_tiling_on_sc=False),
    )(_unit_body)


def _unit_body(xyzT_hbm, src_hbm, dst_hbm, u_hbm,
               xyz_v, src_v, dst_v, ux_v, uy_v, uz_v):
    w = _wid()
    ebase = w * E_SLICE
    pltpu.sync_copy(xyzT_hbm, xyz_v)
    pltpu.sync_copy(src_hbm.at[pl.ds(ebase, E_SLICE)], src_v)
    pltpu.sync_copy(dst_hbm.at[pl.ds(ebase, E_SLICE)], dst_v)

    o1 = jnp.full((L,), N_CG_PAD, jnp.int32)
    o2 = jnp.full((L,), 2 * N_CG_PAD, jnp.int32)

    @plsc.parallel_loop(0, E_SLICE // L, unroll=4)
    def grp_body(i):
        sl = pl.ds(i * L, L)
        s = src_v[sl]
        d = dst_v[sl]
        xs = plsc.load_gather(xyz_v, [s])
        ys = plsc.load_gather(xyz_v, [s + o1])
        zs = plsc.load_gather(xyz_v, [s + o2])
        xd = plsc.load_gather(xyz_v, [d])
        yd = plsc.load_gather(xyz_v, [d + o1])
        zd = plsc.load_gather(xyz_v, [d + o2])
        rx = xd - xs
        ry = yd - ys
        rz = zd - zs
        s2 = rx * rx + ry * ry + rz * rz
        rs = _fast_rsqrt(jnp.maximum(s2, 1e-30))
        dn = s2 * rs
        inv = 1.0 / (dn + 1e-8)
        ux_v[sl] = rx * inv
        uy_v[sl] = ry * inv
        uz_v[sl] = rz * inv

    pltpu.sync_copy(ux_v, u_hbm.at[0, pl.ds(ebase, E_SLICE)])
    pltpu.sync_copy(uy_v, u_hbm.at[1, pl.ds(ebase, E_SLICE)])
    pltpu.sync_copy(uz_v, u_hbm.at[2, pl.ds(ebase, E_SLICE)])


@functools.cache
def _build_edge_kernel():
    mesh = plsc.VectorSubcoreMesh(core_axis_name="c", subcore_axis_name="s")
    buf = lambda dt: pltpu.VMEM((E_CHUNK,), dt)
    return functools.partial(
        pl.kernel,
        mesh=mesh,
        out_type=jax.ShapeDtypeStruct((NW, 3 * N_CG_PAD), jnp.float32),
        scratch_types=[
            pltpu.VMEM((N_CG_PAD,), jnp.float32),
            pltpu.VMEM((3 * N_CG_PAD,), jnp.float32),
            buf(jnp.int32), buf(jnp.int32),
            buf(jnp.float32), buf(jnp.float32), buf(jnp.float32),
            buf(jnp.int32), buf(jnp.int32),
            buf(jnp.float32), buf(jnp.float32), buf(jnp.float32),
            pltpu.SemaphoreType.DMA,
            pltpu.SemaphoreType.DMA,
        ],
        compiler_params=pltpu.CompilerParams(
            needs_layout_passes=False, use_tc_tiling_on_sc=False),
    )(_edge_body)


def _edge_body(gT_hbm, src_hbm, dst_hbm, u_hbm, v_hbm,
               g_v, acc_v,
               sa_v, da_v, uxa_v, uya_v, uza_v,
               sb_v, db_v, uxb_v, uyb_v, uzb_v,
               sem_a, sem_b):
    w = _wid()
    set_a = (sa_v, da_v, uxa_v, uya_v, uza_v)
    set_b = (sb_v, db_v, uxb_v, uyb_v, uzb_v)

    def chunk_copies(g, bufs, sem):
        base = g * E_CHUNK
        sl = pl.ds(base, E_CHUNK)
        return [
            pltpu.make_async_copy(src_hbm.at[sl], bufs[0], sem),
            pltpu.make_async_copy(dst_hbm.at[sl], bufs[1], sem),
            pltpu.make_async_copy(u_hbm.at[0, sl], bufs[2], sem),
            pltpu.make_async_copy(u_hbm.at[1, sl], bufs[3], sem),
            pltpu.make_async_copy(u_hbm.at[2, sl], bufs[4], sem),
        ]

    def start_chunk(g, bufs, sem):
        for cp in chunk_copies(g, bufs, sem):
            cp.start()

    def wait_chunk(g, bufs, sem):
        for cp in chunk_copies(g, bufs, sem):
            cp.wait()

    pltpu.sync_copy(gT_hbm.at[w], g_v)

    zeros16 = jnp.zeros((L,), jnp.float32)

    @plsc.parallel_loop(0, 3 * N_CG_PAD // L, unroll=8)
    def zero_body(i):
        acc_v[pl.ds(i * L, L)] = zeros16

    o1 = jnp.full((L,), N_CG_PAD, jnp.int32)
    o2 = jnp.full((L,), 2 * N_CG_PAD, jnp.int32)

    def process(bufs):
        bsrc, bdst, bux, buy, buz = bufs

        @plsc.parallel_loop(0, E_CHUNK // L, unroll=4)
        def grp_body(i):
            sl = pl.ds(i * L, L)
            s = bsrc[sl]
            d = bdst[sl]
            gv = plsc.load_gather(g_v, [s])
            plsc.addupdate_scatter(acc_v, [d], gv * bux[sl])
            plsc.addupdate_scatter(acc_v, [d + o1], gv * buy[sl])
            plsc.addupdate_scatter(acc_v, [d + o2], gv * buz[sl])

    start_chunk(0, set_a, sem_a)

    def chunk_body(p, carry):
        ga = 2 * p
        wait_chunk(ga, set_a, sem_a)
        start_chunk(ga + 1, set_b, sem_b)
        process(set_a)
        wait_chunk(ga + 1, set_b, sem_b)

        @pl.when(ga + 2 < N_E_CHUNKS)
        def _():
            start_chunk(ga + 2, set_a, sem_a)

        process(set_b)
        return carry

    lax.fori_loop(0, N_E_CHUNKS // 2, chunk_body, 0)
    pltpu.sync_copy(acc_v, v_hbm.at[w])


@functools.cache
def _build_atom_kernel():
    mesh = plsc.VectorSubcoreMesh(core_axis_name="c", subcore_axis_name="s")
    return functools.partial(
        pl.kernel,
        mesh=mesh,
        out_type=jax.ShapeDtypeStruct((NW, 3, A_PER_W), jnp.float32),
        scratch_types=[
            pltpu.VMEM((3 * N_CG_PAD,), jnp.float32),
            pltpu.VMEM((A_PER_W + 32,), jnp.int32),
            pltpu.VMEM((A_PER_W,), jnp.int32),
            pltpu.VMEM((A_PER_W,), jnp.int32),
            pltpu.VMEM((A_PER_W,), jnp.int32),
            pltpu.VMEM((A_PER_W,), jnp.float32),
            pltpu.VMEM((A_PER_W,), jnp.float32),
            pltpu.VMEM((A_PER_W,), jnp.float32),
            pltpu.VMEM((A_PER_W,), jnp.float32),
            pltpu.VMEM((A_PER_W,), jnp.float32),
            pltpu.VMEM((A_PER_W,), jnp.float32),
            pltpu.VMEM((CA_PAD,), jnp.int32),
            pltpu.SemaphoreType.DMA,
        ],
        compiler_params=pltpu.CompilerParams(
            needs_layout_passes=False, use_tc_tiling_on_sc=False),
    )(_atom_body)


def _atom_body(xyzT_hbm, mext_hbm, v4_hbm, ca_hbm, out_hbm,
               xyz_v, m_v, rix_v, riy_v, riz_v, vrx_v, vry_v, vrz_v,
               ox_v, oy_v, oz_v, ca_v, sem):
    w = _wid()
    abase = w * A_PER_W
    pltpu.sync_copy(xyzT_hbm, xyz_v)
    pltpu.sync_copy(mext_hbm.at[pl.ds(abase, A_PER_W + 32)], m_v)
    pltpu.sync_copy(ca_hbm, ca_v)

    iota = lax.iota(jnp.int32, L)
    o1 = jnp.full((L,), N_CG_PAD, jnp.int32)
    o2 = jnp.full((L,), 2 * N_CG_PAD, jnp.int32)

    @plsc.parallel_loop(0, A_PER_W // L, unroll=2)
    def pass1(i):
        q = i * L
        mcur = m_v[pl.ds(32 + q, L)]
        ch = jnp.zeros((L,), jnp.int32)
        for j in range(1, 32):
            mj = plsc.load_gather(m_v, [iota + (32 + q - j)])
            ch = ch + (mj == mcur).astype(jnp.int32)
        r0 = ch * (3 * N_CG_PAD) + mcur
        sl = pl.ds(q, L)
        rix_v[sl] = r0
        riy_v[sl] = r0 + o1
        riz_v[sl] = r0 + o2

    copies = []
    for ridx, vdst in ((rix_v, vrx_v), (riy_v, vry_v), (riz_v, vrz_v)):
        for j in range(A_N_GATHERS):
            sl = pl.ds(j * A_GATHER_ROWS, A_GATHER_ROWS)
            copies.append(
                pltpu.async_copy(v4_hbm.at[ridx.at[sl]], vdst.at[sl], sem))
    for cp in copies:
        cp.wait()

    @plsc.parallel_loop(0, A_PER_W // L, unroll=4)
    def pass2(i):
        q = i * L
        sl = pl.ds(q, L)
        mcur = m_v[pl.ds(32 + q, L)]
        bx = plsc.load_gather(xyz_v, [mcur])
        by = plsc.load_gather(xyz_v, [mcur + o1])
        bz = plsc.load_gather(xyz_v, [mcur + o2])
        ox_v[sl] = vrx_v[sl] + bx
        oy_v[sl] = vry_v[sl] + by
        oz_v[sl] = vrz_v[sl] + bz

    @plsc.parallel_loop(0, CA_PAD // L, unroll=4)
    def pass3(i):
        t = ca_v[pl.ds(i * L, L)]
        lt = t - abase
        inb = (lt >= 0) & (lt < A_PER_W)
        lsafe = jnp.where(inb, lt, 0)
        mt = plsc.load_gather(m_v, [lsafe + 32])
        bx = plsc.load_gather(xyz_v, [mt])
        by = plsc.load_gather(xyz_v, [mt + o1])
        bz = plsc.load_gather(xyz_v, [mt + o2])
        plsc.store_scatter(ox_v, [lsafe], bx, mask=inb)
        plsc.store_scatter(oy_v, [lsafe], by, mask=inb)
        plsc.store_scatter(oz_v, [lsafe], bz, mask=inb)

    pltpu.sync_copy(ox_v, out_hbm.at[w, 0])
    pltpu.sync_copy(oy_v, out_hbm.at[w, 1])
    pltpu.sync_copy(oz_v, out_hbm.at[w, 2])


def kernel(cg_z, xyz, cg_xyz, bond_edge_list, CG_nbr_list, cg_map, ca_idx,
           emb, W1, Wg):
    f32 = jnp.float32
    i32 = jnp.int32

    cz_pad = jnp.zeros((N_CG_PAD,), i32).at[:N_CG].set(cg_z.astype(i32))
    embT_pad = jnp.zeros((D, D), f32).at[:, :emb.shape[0]].set(emb.T)
    gT = _gate_features(cz_pad, embT_pad, W1.T, Wg.T)

    xyzT = jnp.zeros((3, N_CG_PAD), f32).at[:, :N_CG].set(cg_xyz.T)
    xyzT_flat = xyzT.reshape(3 * N_CG_PAD)
    src = CG_nbr_list[:, 0].astype(i32)
    dst = CG_nbr_list[:, 1].astype(i32)

    u = _build_unit_kernel()(xyzT_flat, src, dst)
    v_flat = _build_edge_kernel()(gT, src, dst, u)
    v4 = v_flat.reshape(NW * 3 * N_CG_PAD)

    mext = jnp.concatenate([
        jnp.full((32,), -1, i32),
        cg_map.astype(i32),
        jnp.zeros((N_ATOM_PAD - N_ATOM,), i32),
    ])
    ca_pad = jnp.concatenate([
        ca_idx.astype(i32),
        jnp.full((CA_PAD - ca_idx.shape[0],), -1, i32),
    ])

    out = _build_atom_kernel()(xyzT_flat, mext, v4, ca_pad)
    xyz_recon = out.transpose(0, 2, 1).reshape(N_ATOM_PAD, 3)[:N_ATOM]
    return (xyz, xyz_recon)


# speedup vs baseline: 146.9756x; 1.0612x over previous
import functools

import jax
import jax.numpy as jnp
from jax import lax
from jax.experimental import pallas as pl
from jax.experimental.pallas import tpu as pltpu
from jax.experimental.pallas import tpu_sc as plsc

N_CG = 10000
N_CG_PAD = 10240
N_ATOM = 80000
N_ATOM_PAD = 81920
E_CG = 320000
D = 128
C = 32

NC = 2
NS = 16
NW = NC * NS
L = 16

BN = 1280
E_CHUNK = 8000
N_E_CHUNKS = E_CG // E_CHUNK
A_PER_W = N_ATOM_PAD // NW
A_GATHER_ROWS = 128
A_N_GATHERS = A_PER_W // A_GATHER_ROWS
CA_PAD = 10240


def _gate_body(cz_ref, embT_ref, w1T_ref, wgT_ref, out_ref):
    cz = cz_ref[0, 0, :]
    rows = lax.broadcasted_iota(jnp.int32, (D, BN), 0)
    oh = (rows == cz[None, :]).astype(jnp.float32)
    st = jnp.dot(embT_ref[...], oh, preferred_element_type=jnp.float32)
    ht = jnp.maximum(
        jnp.dot(w1T_ref[...], st, preferred_element_type=jnp.float32), 0.0)
    out_ref[...] = jnp.dot(wgT_ref[...], ht,
                           preferred_element_type=jnp.float32)


def _gate_features(cz_pad, embT_pad, w1T, wgT):
    n_blocks = N_CG_PAD // BN
    czp = cz_pad.reshape(n_blocks, 1, BN)
    return pl.pallas_call(
        _gate_body,
        grid=(n_blocks,),
        in_specs=[
            pl.BlockSpec((1, 1, BN), lambda i: (i, 0, 0)),
            pl.BlockSpec((D, D), lambda i: (0, 0)),
            pl.BlockSpec((D, D), lambda i: (0, 0)),
            pl.BlockSpec((C, D), lambda i: (0, 0)),
        ],
        out_specs=pl.BlockSpec((C, BN), lambda i: (0, i)),
        out_shape=jax.ShapeDtypeStruct((C, N_CG_PAD), jnp.float32),
    )(czp, embT_pad, w1T, wgT)


def _fast_rsqrt(x):
    i = plsc.bitcast(x, jnp.int32)
    y = plsc.bitcast(jnp.int32(0x5F3759DF) - (i >> 1), jnp.float32)
    h = x * 0.5
    y = y * (1.5 - h * y * y)
    y = y * (1.5 - h * y * y)
    y = y * (1.5 - h * y * y)
    return y


def _wid():
    return lax.axis_index("s") * NC + lax.axis_index("c")


E_SLICE = E_CG // NW


@functools.cache
def _build_unit_kernel():
    mesh = plsc.VectorSubcoreMesh(core_axis_name="c", subcore_axis_name="s")
    return functools.partial(
        pl.kernel,
        mesh=mesh,
        out_type=jax.ShapeDtypeStruct((3, E_CG), jnp.float32),
        scratch_types=[
            pltpu.VMEM((3 * N_CG_PAD,), jnp.float32),
            pltpu.VMEM((E_SLICE,), jnp.int32),
            pltpu.VMEM((E_SLICE,), jnp.float32),
            pltpu.VMEM((E_SLICE,), jnp.float32),
            pltpu.VMEM((E_SLICE,), jnp.float32),
        ],
        compiler_params=pltpu.CompilerParams(
            needs_layout_passes=False, use_tc_tiling_on_sc=False),
    )(_unit_body)


def _unit_body(xyzT_hbm, pk_hbm, u_hbm,
               xyz_v, pk_v, ux_v, uy_v, uz_v):
    w = _wid()
    ebase = w * E_SLICE
    pltpu.sync_copy(xyzT_hbm, xyz_v)
    pltpu.sync_copy(pk_hbm.at[pl.ds(ebase, E_SLICE)], pk_v)

    o1 = jnp.full((L,), N_CG_PAD, jnp.int32)
    o2 = jnp.full((L,), 2 * N_CG_PAD, jnp.int32)
    m16 = jnp.full((L,), 0xFFFF, jnp.int32)

    @plsc.parallel_loop(0, E_SLICE // L, unroll=4)
    def grp_body(i):
        sl = pl.ds(i * L, L)
        p = pk_v[sl]
        s = p & m16
        d = p >> 16
        xs = plsc.load_gather(xyz_v, [s])
        ys = plsc.load_gather(xyz_v, [s + o1])
        zs = plsc.load_gather(xyz_v, [s + o2])
        xd = plsc.load_gather(xyz_v, [d])
        yd = plsc.load_gather(xyz_v, [d + o1])
        zd = plsc.load_gather(xyz_v, [d + o2])
        rx = xd - xs
        ry = yd - ys
        rz = zd - zs
        s2 = rx * rx + ry * ry + rz * rz
        rs = _fast_rsqrt(jnp.maximum(s2, 1e-30))
        dn = s2 * rs
        inv = 1.0 / (dn + 1e-8)
        ux_v[sl] = rx * inv
        uy_v[sl] = ry * inv
        uz_v[sl] = rz * inv

    pltpu.sync_copy(ux_v, u_hbm.at[0, pl.ds(ebase, E_SLICE)])
    pltpu.sync_copy(uy_v, u_hbm.at[1, pl.ds(ebase, E_SLICE)])
    pltpu.sync_copy(uz_v, u_hbm.at[2, pl.ds(ebase, E_SLICE)])


@functools.cache
def _build_edge_kernel():
    mesh = plsc.VectorSubcoreMesh(core_axis_name="c", subcore_axis_name="s")
    buf = lambda dt: pltpu.VMEM((E_CHUNK,), dt)
    return functools.partial(
        pl.kernel,
        mesh=mesh,
        out_type=jax.ShapeDtypeStruct((NW, 3 * N_CG_PAD), jnp.float32),
        scratch_types=[
            pltpu.VMEM((N_CG_PAD,), jnp.float32),
            pltpu.VMEM((3 * N_CG_PAD,), jnp.float32),
            buf(jnp.int32),
            buf(jnp.float32), buf(jnp.float32), buf(jnp.float32),
            buf(jnp.int32),
            buf(jnp.float32), buf(jnp.float32), buf(jnp.float32),
            pltpu.SemaphoreType.DMA,
            pltpu.SemaphoreType.DMA,
        ],
        compiler_params=pltpu.CompilerParams(
            needs_layout_passes=False, use_tc_tiling_on_sc=False),
    )(_edge_body)


def _edge_body(gT_hbm, pk_hbm, u_hbm, v_hbm,
               g_v, acc_v,
               pa_v, uxa_v, uya_v, uza_v,
               pb_v, uxb_v, uyb_v, uzb_v,
               sem_a, sem_b):
    w = _wid()
    set_a = (pa_v, uxa_v, uya_v, uza_v)
    set_b = (pb_v, uxb_v, uyb_v, uzb_v)

    def chunk_copies(g, bufs, sem):
        base = g * E_CHUNK
        sl = pl.ds(base, E_CHUNK)
        return [
            pltpu.make_async_copy(pk_hbm.at[sl], bufs[0], sem),
            pltpu.make_async_copy(u_hbm.at[0, sl], bufs[1], sem),
            pltpu.make_async_copy(u_hbm.at[1, sl], bufs[2], sem),
            pltpu.make_async_copy(u_hbm.at[2, sl], bufs[3], sem),
        ]

    def start_chunk(g, bufs, sem):
        for cp in chunk_copies(g, bufs, sem):
            cp.start()

    def wait_chunk(g, bufs, sem):
        for cp in chunk_copies(g, bufs, sem):
            cp.wait()

    pltpu.sync_copy(gT_hbm.at[w], g_v)

    zeros16 = jnp.zeros((L,), jnp.float32)

    @plsc.parallel_loop(0, 3 * N_CG_PAD // L, unroll=8)
    def zero_body(i):
        acc_v[pl.ds(i * L, L)] = zeros16

    o1 = jnp.full((L,), N_CG_PAD, jnp.int32)
    o2 = jnp.full((L,), 2 * N_CG_PAD, jnp.int32)
    m16 = jnp.full((L,), 0xFFFF, jnp.int32)

    def process(bufs):
        bpk, bux, buy, buz = bufs

        @plsc.parallel_loop(0, E_CHUNK // L, unroll=4)
        def grp_body(i):
            sl = pl.ds(i * L, L)
            p = bpk[sl]
            s = p & m16
            d = p >> 16
            gv = plsc.load_gather(g_v, [s])
            plsc.addupdate_scatter(acc_v, [d], gv * bux[sl])
            plsc.addupdate_scatter(acc_v, [d + o1], gv * buy[sl])
            plsc.addupdate_scatter(acc_v, [d + o2], gv * buz[sl])

    start_chunk(0, set_a, sem_a)

    def chunk_body(p, carry):
        ga = 2 * p
        wait_chunk(ga, set_a, sem_a)
        start_chunk(ga + 1, set_b, sem_b)
        process(set_a)
        wait_chunk(ga + 1, set_b, sem_b)

        @pl.when(ga + 2 < N_E_CHUNKS)
        def _():
            start_chunk(ga + 2, set_a, sem_a)

        process(set_b)
        return carry

    lax.fori_loop(0, N_E_CHUNKS // 2, chunk_body, 0)
    pltpu.sync_copy(acc_v, v_hbm.at[w])


@functools.cache
def _build_atom_kernel():
    mesh = plsc.VectorSubcoreMesh(core_axis_name="c", subcore_axis_name="s")
    return functools.partial(
        pl.kernel,
        mesh=mesh,
        out_type=jax.ShapeDtypeStruct((NW, 3, A_PER_W), jnp.float32),
        scratch_types=[
            pltpu.VMEM((3 * N_CG_PAD,), jnp.float32),
            pltpu.VMEM((A_PER_W + 32,), jnp.int32),
            pltpu.VMEM((A_PER_W,), jnp.int32),
            pltpu.VMEM((A_PER_W,), jnp.int32),
            pltpu.VMEM((A_PER_W,), jnp.int32),
            pltpu.VMEM((A_PER_W,), jnp.float32),
            pltpu.VMEM((A_PER_W,), jnp.float32),
            pltpu.VMEM((A_PER_W,), jnp.float32),
            pltpu.VMEM((A_PER_W,), jnp.float32),
            pltpu.VMEM((A_PER_W,), jnp.float32),
            pltpu.VMEM((A_PER_W,), jnp.float32),
            pltpu.VMEM((CA_PAD,), jnp.int32),
            pltpu.SemaphoreType.DMA,
        ],
        compiler_params=pltpu.CompilerParams(
            needs_layout_passes=False, use_tc_tiling_on_sc=False),
    )(_atom_body)


def _atom_body(xyzT_hbm, mext_hbm, v4_hbm, ca_hbm, out_hbm,
               xyz_v, m_v, rix_v, riy_v, riz_v, vrx_v, vry_v, vrz_v,
               ox_v, oy_v, oz_v, ca_v, sem):
    w = _wid()
    abase = w * A_PER_W
    pltpu.sync_copy(xyzT_hbm, xyz_v)
    pltpu.sync_copy(mext_hbm.at[pl.ds(abase, A_PER_W + 32)], m_v)
    pltpu.sync_copy(ca_hbm, ca_v)

    iota = lax.iota(jnp.int32, L)
    o1 = jnp.full((L,), N_CG_PAD, jnp.int32)
    o2 = jnp.full((L,), 2 * N_CG_PAD, jnp.int32)

    @plsc.parallel_loop(0, A_PER_W // L, unroll=2)
    def pass1(i):
        q = i * L
        mcur = m_v[pl.ds(32 + q, L)]
        ch = jnp.zeros((L,), jnp.int32)
        for j in range(1, 32):
            mj = plsc.load_gather(m_v, [iota + (32 + q - j)])
            ch = ch + (mj == mcur).astype(jnp.int32)
        r0 = ch * (3 * N_CG_PAD) + mcur
        sl = pl.ds(q, L)
        rix_v[sl] = r0
        riy_v[sl] = r0 + o1
        riz_v[sl] = r0 + o2

    copies = []
    for ridx, vdst in ((rix_v, vrx_v), (riy_v, vry_v), (riz_v, vrz_v)):
        for j in range(A_N_GATHERS):
            sl = pl.ds(j * A_GATHER_ROWS, A_GATHER_ROWS)
            copies.append(
                pltpu.async_copy(v4_hbm.at[ridx.at[sl]], vdst.at[sl], sem))
    for cp in copies:
        cp.wait()

    @plsc.parallel_loop(0, A_PER_W // L, unroll=4)
    def pass2(i):
        q = i * L
        sl = pl.ds(q, L)
        mcur = m_v[pl.ds(32 + q, L)]
        bx = plsc.load_gather(xyz_v, [mcur])
        by = plsc.load_gather(xyz_v, [mcur + o1])
        bz = plsc.load_gather(xyz_v, [mcur + o2])
        ox_v[sl] = vrx_v[sl] + bx
        oy_v[sl] = vry_v[sl] + by
        oz_v[sl] = vrz_v[sl] + bz

    @plsc.parallel_loop(0, CA_PAD // L, unroll=4)
    def pass3(i):
        t = ca_v[pl.ds(i * L, L)]
        lt = t - abase
        inb = (lt >= 0) & (lt < A_PER_W)
        lsafe = jnp.where(inb, lt, 0)
        mt = plsc.load_gather(m_v, [lsafe + 32])
        bx = plsc.load_gather(xyz_v, [mt])
        by = plsc.load_gather(xyz_v, [mt + o1])
        bz = plsc.load_gather(xyz_v, [mt + o2])
        plsc.store_scatter(ox_v, [lsafe], bx, mask=inb)
        plsc.store_scatter(oy_v, [lsafe], by, mask=inb)
        plsc.store_scatter(oz_v, [lsafe], bz, mask=inb)

    pltpu.sync_copy(ox_v, out_hbm.at[w, 0])
    pltpu.sync_copy(oy_v, out_hbm.at[w, 1])
    pltpu.sync_copy(oz_v, out_hbm.at[w, 2])


def kernel(cg_z, xyz, cg_xyz, bond_edge_list, CG_nbr_list, cg_map, ca_idx,
           emb, W1, Wg):
    f32 = jnp.float32
    i32 = jnp.int32

    cz_pad = jnp.zeros((N_CG_PAD,), i32).at[:N_CG].set(cg_z.astype(i32))
    embT_pad = jnp.zeros((D, D), f32).at[:, :emb.shape[0]].set(emb.T)
    gT = _gate_features(cz_pad, embT_pad, W1.T, Wg.T)

    xyzT = jnp.zeros((3, N_CG_PAD), f32).at[:, :N_CG].set(cg_xyz.T)
    xyzT_flat = xyzT.reshape(3 * N_CG_PAD)
    src = CG_nbr_list[:, 0].astype(i32)
    dst = CG_nbr_list[:, 1].astype(i32)
    pk = (dst << 16) | src

    u = _build_unit_kernel()(xyzT_flat, pk)
    v_flat = _build_edge_kernel()(gT, pk, u)
    v4 = v_flat.reshape(NW * 3 * N_CG_PAD)

    mext = jnp.concatenate([
        jnp.full((32,), -1, i32),
        cg_map.astype(i32),
        jnp.zeros((N_ATOM_PAD - N_ATOM,), i32),
    ])
    ca_pad = jnp.concatenate([
        ca_idx.astype(i32),
        jnp.full((CA_PAD - ca_idx.shape[0],), -1, i32),
    ])

    out = _build_atom_kernel()(xyzT_flat, mext, v4, ca_pad)
    xyz_recon = out.transpose(0, 2, 1).reshape(N_ATOM_PAD, 3)[:N_ATOM]
    return (xyz, xyz_recon)
